# baseline (speedup 1.0000x reference)
import functools

import jax
import jax.numpy as jnp
from jax import lax
from jax.experimental import pallas as pl
from jax.experimental.pallas import tpu as pltpu
from jax.experimental.pallas import tpu_sc as plsc

_NL0, _NL1 = 16000, 4000
_P0, _P1 = 16384, 4096
_K = 2000
_TOPC = 2048
_OUTP = 1024
_POST = 1000
_IOU_T = 0.7
_MIN_SZ = 0.001
_IMG = 800.0
_BLK = 256
_NCAND = 2 * _TOPC
_TBL = _P0 + _P1
_TBLD = 128
_NW = 32
_BPW = _NCAND // _NW


def _partner(x, d):
    r, c = x.shape
    if d < c:
        lo = jnp.roll(x, -d, axis=1)
        hi = jnp.roll(x, d, axis=1)
        col = jax.lax.broadcasted_iota(jnp.int32, x.shape, 1)
        return jnp.where((col & d) == 0, lo, hi)
    dr = d // c
    lo = jnp.roll(x, -dr, axis=0)
    hi = jnp.roll(x, dr, axis=0)
    row = jax.lax.broadcasted_iota(jnp.int32, x.shape, 0)
    return jnp.where((row & dr) == 0, lo, hi)


def _cmp_swap(kv, ki, extras, d, take_winner):
    pv = _partner(kv, d)
    pi = _partner(ki, d)
    self_first = (kv > pv) | ((kv == pv) & (ki < pi))
    keep_self = self_first == take_winner
    outv = jnp.where(keep_self, kv, pv)
    outi = jnp.where(keep_self, ki, pi)
    outx = [jnp.where(keep_self, e, _partner(e, d)) for e in extras]
    return outv, outi, outx


def _flat_iota(shape):
    r = jax.lax.broadcasted_iota(jnp.int32, shape, 0)
    c = jax.lax.broadcasted_iota(jnp.int32, shape, 1)
    return r * shape[1] + c


def _bitonic_sort_desc(kv, ki, extras=()):
    n = kv.shape[0] * kv.shape[1]
    flat = _flat_iota(kv.shape)
    extras = list(extras)
    k = 2
    while k <= n:
        desc_blk = (flat & k) == 0
        j = k >> 1
        while j:
            lower = (flat & j) == 0
            tw = lower == desc_blk
            kv, ki, extras = _cmp_swap(kv, ki, extras, j, tw)
            j >>= 1
        k <<= 1
    return kv, ki, extras


def _iou_gt(ax1, ay1, ax2, ay2, aarea, bx1, by1, bx2, by2, barea):
    ix1 = jnp.maximum(ax1, bx1)
    iy1 = jnp.maximum(ay1, by1)
    ix2 = jnp.minimum(ax2, bx2)
    iy2 = jnp.minimum(ay2, by2)
    inter = jnp.maximum(ix2 - ix1, 0.0) * jnp.maximum(iy2 - iy1, 0.0)
    iou = inter / (aarea + barea - inter + 1e-9)
    return iou > _IOU_T


def _nms_level(vf, g):
    x1 = jnp.clip(g[0:1, :], 0.0, _IMG)
    y1 = jnp.clip(g[1:2, :], 0.0, _IMG)
    x2 = jnp.clip(g[2:3, :], 0.0, _IMG)
    y2 = jnp.clip(g[3:4, :], 0.0, _IMG)
    score = jax.nn.sigmoid(vf)

    rank = jax.lax.broadcasted_iota(jnp.int32, (1, _TOPC), 1)
    valid = ((x2 - x1 >= _MIN_SZ) & (y2 - y1 >= _MIN_SZ)
             & (score >= 0.0) & (rank < _K))
    ns = jnp.where(valid, score, -jnp.inf)
    area = (x2 - x1) * (y2 - y1)

    sup = jnp.zeros((1, _TOPC), jnp.bool_)
    kept_blocks = []
    tri = (jax.lax.broadcasted_iota(jnp.int32, (_BLK, _BLK), 0)
           < jax.lax.broadcasted_iota(jnp.int32, (_BLK, _BLK), 1))
    for b in range(_TOPC // _BLK):
        sl = slice(b * _BLK, (b + 1) * _BLK)
        valid_blk = (ns[:, sl] > -jnp.inf) & ~sup[:, sl]
        bx1, by1 = x1[:, sl], y1[:, sl]
        bx2, by2 = x2[:, sl], y2[:, sl]
        ba = area[:, sl]
        tx1, ty1 = bx1.reshape(_BLK, 1), by1.reshape(_BLK, 1)
        tx2, ty2 = bx2.reshape(_BLK, 1), by2.reshape(_BLK, 1)
        ta = ba.reshape(_BLK, 1)
        s_blk = _iou_gt(tx1, ty1, tx2, ty2, ta, bx1, by1, bx2, by2, ba) & tri
        s_f = s_blk.astype(jnp.float32)

        def cond(st):
            return st[1]

        def body(st):
            kp = st[0]
            supv = jnp.dot(kp, s_f, preferred_element_type=jnp.float32) > 0.5
            kn = (valid_blk & ~supv).astype(jnp.float32)
            return kn, jnp.any(kn != kp)

        keep_f, _ = jax.lax.while_loop(
            cond, body, (valid_blk.astype(jnp.float32), True))
        keep = keep_f > 0.5
        kept_blocks.append(keep)
        lo = (b + 1) * _BLK
        if lo < _TOPC:
            rest = slice(lo, _TOPC)
            hits = _iou_gt(tx1, ty1, tx2, ty2, ta, x1[:, rest], y1[:, rest],
                           x2[:, rest], y2[:, rest], area[:, rest])
            supadd = jnp.dot(keep_f, hits.astype(jnp.float32),
                             preferred_element_type=jnp.float32) > 0.5
            pre = jnp.zeros((1, lo), jnp.bool_)
            sup = sup | jnp.concatenate([pre, supadd], axis=1)
    kept = jnp.concatenate(kept_blocks, axis=1)
    return score, x1, y1, x2, y2, rank, kept


def _sort_body(obj0_ref, obj1_ref, sv_ref, si_ref):
    sv0, si0, _ = _bitonic_sort_desc(obj0_ref[...], _flat_iota(obj0_ref.shape))
    sv1, si1, _ = _bitonic_sort_desc(obj1_ref[...], _flat_iota(obj1_ref.shape))
    sv_ref[...] = jnp.concatenate([sv0[0:1, :], sv1[0:1, :]], axis=0)
    si_ref[...] = jnp.concatenate([si0[0:1, :], si1[0:1, :] + _P0], axis=0)


_SORT_CALL = pl.pallas_call(
    _sort_body,
    out_shape=(jax.ShapeDtypeStruct((2, _TOPC), jnp.float32),
               jax.ShapeDtypeStruct((2, _TOPC), jnp.int32)),
)


@functools.partial(
    pl.kernel,
    mesh=plsc.VectorSubcoreMesh(core_axis_name="c", subcore_axis_name="s"),
    out_type=jax.ShapeDtypeStruct((_NCAND, _TBLD), jnp.float32),
    scratch_types=[
        pltpu.VMEM((_BPW,), jnp.int32),
        pltpu.VMEM((_BPW, _TBLD), jnp.float32),
        pltpu.SemaphoreType.DMA,
    ],
)
def _sc_gather(table_hbm, idx_hbm, out_hbm, idx_v, rows_v, sem):
    wid = lax.axis_index("s") * 2 + lax.axis_index("c")
    base = wid * _BPW
    pltpu.sync_copy(idx_hbm.at[pl.ds(base, _BPW)], idx_v)
    pltpu.async_copy(table_hbm.at[idx_v], rows_v, sem).wait()
    pltpu.sync_copy(rows_v, out_hbm.at[pl.ds(base, _BPW)])


def _post_body(sv_ref, g0_ref, g1_ref, out_ref):
    sv = sv_ref[...]
    s0, x10, y10, x20, y20, r0, k0 = _nms_level(sv[0:1, :], g0_ref[...])
    s1, x11, y11, x21, y21, r1, k1 = _nms_level(sv[1:2, :], g1_ref[...])

    v0 = jnp.where(k0, s0, -jnp.inf)
    v1 = jnp.where(k1, s1, -jnp.inf)
    slot0 = r0
    slot1 = r1 + _K

    def cat(a, b):
        return jnp.concatenate([a, b], axis=0)

    mv = cat(v0, v1)
    mi = cat(slot0, slot1)
    ext = [cat(x10, x11), cat(y10, y11), cat(x20, x21), cat(y20, y21)]
    mv, mi, ext = _bitonic_sort_desc(mv, mi, ext)

    ok = mv[0:1, :_OUTP] > -jnp.inf
    rows = [jnp.where(ok, e[0:1, :_OUTP], 0.0) for e in ext]
    rows.append(jnp.where(ok, mv[0:1, :_OUTP], 0.0))
    out_ref[...] = jnp.concatenate(rows, axis=0)


_POST_CALL = pl.pallas_call(
    _post_body,
    out_shape=jax.ShapeDtypeStruct((5, _OUTP), jnp.float32),
)


def kernel(proposals, objectness):
    objectness = jax.lax.stop_gradient(objectness)
    o0 = jnp.pad(objectness[:_NL0], (0, _P0 - _NL0),
                 constant_values=-jnp.inf).reshape(8, _P0 // 8)
    o1 = jnp.pad(objectness[_NL0:], (0, _P1 - _NL1),
                 constant_values=-jnp.inf).reshape(2, _P1 // 2)
    table = jnp.concatenate(
        [jnp.pad(proposals[:_NL0], ((0, _P0 - _NL0), (0, 0))),
         jnp.pad(proposals[_NL0:], ((0, _P1 - _NL1), (0, 0)))], axis=0)
    table = jnp.pad(table, ((0, 0), (0, _TBLD - 4)))
    sv, si = _SORT_CALL(o0, o1)
    idx = si.reshape(_NCAND)
    g = _sc_gather(table, idx)
    g0 = g[:_TOPC, :4].T
    g1 = g[_TOPC:, :4].T
    out5 = _POST_CALL(sv, g0, g1)
    return out5.T[:_POST]


# speedup vs baseline: 1.0315x; 1.0315x over previous
import functools

import jax
import jax.numpy as jnp
from jax import lax
from jax.experimental import pallas as pl
from jax.experimental.pallas import tpu as pltpu
from jax.experimental.pallas import tpu_sc as plsc

_NL0, _NL1 = 16000, 4000
_P0, _P1 = 16384, 4096
_K = 2000
_TOPC = 2048
_OUTP = 1024
_POST = 1000
_IOU_T = 0.7
_MIN_SZ = 0.001
_IMG = 800.0
_BLK = 256
_NCAND = 2 * _TOPC
_TBL = _P0 + _P1
_TBLD = 128
_NW = 32
_BPW = _NCAND // _NW


def _partner(x, d):
    r, c = x.shape
    if d < c:
        lo = jnp.roll(x, -d, axis=1)
        hi = jnp.roll(x, d, axis=1)
        col = jax.lax.broadcasted_iota(jnp.int32, x.shape, 1)
        return jnp.where((col & d) == 0, lo, hi)
    dr = d // c
    lo = jnp.roll(x, -dr, axis=0)
    hi = jnp.roll(x, dr, axis=0)
    row = jax.lax.broadcasted_iota(jnp.int32, x.shape, 0)
    return jnp.where((row & dr) == 0, lo, hi)


def _cmp_swap(kv, ki, extras, d, take_winner):
    pv = _partner(kv, d)
    pi = _partner(ki, d)
    self_first = (kv > pv) | ((kv == pv) & (ki < pi))
    keep_self = self_first == take_winner
    outv = jnp.where(keep_self, kv, pv)
    outi = jnp.where(keep_self, ki, pi)
    outx = [jnp.where(keep_self, e, _partner(e, d)) for e in extras]
    return outv, outi, outx


def _flat_iota(shape):
    r = jax.lax.broadcasted_iota(jnp.int32, shape, 0)
    c = jax.lax.broadcasted_iota(jnp.int32, shape, 1)
    return r * shape[1] + c


def _bitonic_sort_desc(kv, ki, extras=()):
    n = kv.shape[0] * kv.shape[1]
    flat = _flat_iota(kv.shape)
    extras = list(extras)
    k = 2
    while k <= n:
        desc_blk = (flat & k) == 0
        j = k >> 1
        while j:
            lower = (flat & j) == 0
            tw = lower == desc_blk
            kv, ki, extras = _cmp_swap(kv, ki, extras, j, tw)
            j >>= 1
        k <<= 1
    return kv, ki, extras


def _iou_gt(ax1, ay1, ax2, ay2, aarea, bx1, by1, bx2, by2, barea):
    ix1 = jnp.maximum(ax1, bx1)
    iy1 = jnp.maximum(ay1, by1)
    ix2 = jnp.minimum(ax2, bx2)
    iy2 = jnp.minimum(ay2, by2)
    inter = jnp.maximum(ix2 - ix1, 0.0) * jnp.maximum(iy2 - iy1, 0.0)
    iou = inter / (aarea + barea - inter + 1e-9)
    return iou > _IOU_T


def _level_prep(vf, g):
    x1 = jnp.clip(g[0:1, :], 0.0, _IMG)
    y1 = jnp.clip(g[1:2, :], 0.0, _IMG)
    x2 = jnp.clip(g[2:3, :], 0.0, _IMG)
    y2 = jnp.clip(g[3:4, :], 0.0, _IMG)
    score = jax.nn.sigmoid(vf)
    rank = jax.lax.broadcasted_iota(jnp.int32, (1, _TOPC), 1)
    valid = ((x2 - x1 >= _MIN_SZ) & (y2 - y1 >= _MIN_SZ)
             & (score >= 0.0) & (rank < _K))
    ns = jnp.where(valid, score, -jnp.inf)
    area = (x2 - x1) * (y2 - y1)
    return score, x1, y1, x2, y2, rank, ns, area


def _nms_two_levels(lv0, lv1):
    tri = (jax.lax.broadcasted_iota(jnp.int32, (_BLK, _BLK), 0)
           < jax.lax.broadcasted_iota(jnp.int32, (_BLK, _BLK), 1))
    zblk = jnp.zeros((_BLK, _BLK), jnp.float32)
    sups = [jnp.zeros((1, _TOPC), jnp.bool_), jnp.zeros((1, _TOPC), jnp.bool_)]
    kept_blocks = [[], []]
    for b in range(_TOPC // _BLK):
        sl = slice(b * _BLK, (b + 1) * _BLK)
        s_fs, valids, txs = [], [], []
        for li, lv in enumerate((lv0, lv1)):
            _, x1, y1, x2, y2, _, ns, area = lv
            valids.append((ns[:, sl] > -jnp.inf) & ~sups[li][:, sl])
            bx1, by1 = x1[:, sl], y1[:, sl]
            bx2, by2 = x2[:, sl], y2[:, sl]
            ba = area[:, sl]
            tx = (bx1.reshape(_BLK, 1), by1.reshape(_BLK, 1),
                  bx2.reshape(_BLK, 1), by2.reshape(_BLK, 1),
                  ba.reshape(_BLK, 1))
            txs.append(tx)
            s_blk = _iou_gt(*tx, bx1, by1, bx2, by2, ba) & tri
            s_fs.append(s_blk.astype(jnp.float32))
        s_j = jnp.concatenate(
            [jnp.concatenate([s_fs[0], zblk], axis=1),
             jnp.concatenate([zblk, s_fs[1]], axis=1)], axis=0)
        valid_j = jnp.concatenate(valids, axis=1)

        def cond(st):
            return st[1]

        def body(st):
            kp = st[0]
            supv = jnp.dot(kp, s_j, preferred_element_type=jnp.float32) > 0.5
            kn = (valid_j & ~supv).astype(jnp.float32)
            return kn, jnp.any(kn != kp)

        keep_f, _ = jax.lax.while_loop(
            cond, body, (valid_j.astype(jnp.float32), True))
        lo = (b + 1) * _BLK
        for li, lv in enumerate((lv0, lv1)):
            kf = keep_f[:, li * _BLK:(li + 1) * _BLK]
            kept_blocks[li].append(kf > 0.5)
            if lo < _TOPC:
                _, x1, y1, x2, y2, _, _, area = lv
                rest = slice(lo, _TOPC)
                hits = _iou_gt(*txs[li], x1[:, rest], y1[:, rest],
                               x2[:, rest], y2[:, rest], area[:, rest])
                supadd = jnp.dot(kf, hits.astype(jnp.float32),
                                 preferred_element_type=jnp.float32) > 0.5
                pre = jnp.zeros((1, lo), jnp.bool_)
                sups[li] = sups[li] | jnp.concatenate([pre, supadd], axis=1)
    return (jnp.concatenate(kept_blocks[0], axis=1),
            jnp.concatenate(kept_blocks[1], axis=1))


def _sort_body(obj0_ref, obj1_ref, sv_ref, si_ref):
    sv0, si0, _ = _bitonic_sort_desc(obj0_ref[...], _flat_iota(obj0_ref.shape))
    sv1, si1, _ = _bitonic_sort_desc(obj1_ref[...], _flat_iota(obj1_ref.shape))
    sv_ref[...] = jnp.concatenate([sv0[0:1, :], sv1[0:1, :]], axis=0)
    si_ref[...] = jnp.concatenate([si0[0:1, :], si1[0:1, :] + _P0], axis=0)


_SORT_CALL = pl.pallas_call(
    _sort_body,
    out_shape=(jax.ShapeDtypeStruct((2, _TOPC), jnp.float32),
               jax.ShapeDtypeStruct((2, _TOPC), jnp.int32)),
)


@functools.partial(
    pl.kernel,
    mesh=plsc.VectorSubcoreMesh(core_axis_name="c", subcore_axis_name="s"),
    out_type=jax.ShapeDtypeStruct((_NCAND, _TBLD), jnp.float32),
    scratch_types=[
        pltpu.VMEM((_BPW,), jnp.int32),
        pltpu.VMEM((_BPW, _TBLD), jnp.float32),
        pltpu.SemaphoreType.DMA,
    ],
)
def _sc_gather(table_hbm, idx_hbm, out_hbm, idx_v, rows_v, sem):
    wid = lax.axis_index("s") * 2 + lax.axis_index("c")
    base = wid * _BPW
    pltpu.sync_copy(idx_hbm.at[pl.ds(base, _BPW)], idx_v)
    pltpu.async_copy(table_hbm.at[idx_v], rows_v, sem).wait()
    pltpu.sync_copy(rows_v, out_hbm.at[pl.ds(base, _BPW)])


def _post_body(sv_ref, g0_ref, g1_ref, out_ref):
    sv = sv_ref[...]
    lv0 = _level_prep(sv[0:1, :], g0_ref[...])
    lv1 = _level_prep(sv[1:2, :], g1_ref[...])
    k0, k1 = _nms_two_levels(lv0, lv1)
    s0, x10, y10, x20, y20, r0 = lv0[:6]
    s1, x11, y11, x21, y21, r1 = lv1[:6]

    v0 = jnp.where(k0, s0, -jnp.inf)
    v1 = jnp.where(k1, s1, -jnp.inf)
    slot0 = r0
    slot1 = r1 + _K

    def cat(a, b):
        return jnp.concatenate([a, b], axis=0)

    mv = cat(v0, v1)
    mi = cat(slot0, slot1)
    ext = [cat(x10, x11), cat(y10, y11), cat(x20, x21), cat(y20, y21)]
    mv, mi, ext = _bitonic_sort_desc(mv, mi, ext)

    ok = mv[0:1, :_OUTP] > -jnp.inf
    rows = [jnp.where(ok, e[0:1, :_OUTP], 0.0) for e in ext]
    rows.append(jnp.where(ok, mv[0:1, :_OUTP], 0.0))
    out_ref[...] = jnp.concatenate(rows, axis=0)


_POST_CALL = pl.pallas_call(
    _post_body,
    out_shape=jax.ShapeDtypeStruct((5, _OUTP), jnp.float32),
)


def kernel(proposals, objectness):
    objectness = jax.lax.stop_gradient(objectness)
    o0 = jnp.pad(objectness[:_NL0], (0, _P0 - _NL0),
                 constant_values=-jnp.inf).reshape(8, _P0 // 8)
    o1 = jnp.pad(objectness[_NL0:], (0, _P1 - _NL1),
                 constant_values=-jnp.inf).reshape(2, _P1 // 2)
    table = jnp.concatenate(
        [jnp.pad(proposals[:_NL0], ((0, _P0 - _NL0), (0, 0))),
         jnp.pad(proposals[_NL0:], ((0, _P1 - _NL1), (0, 0)))], axis=0)
    table = jnp.pad(table, ((0, 0), (0, _TBLD - 4)))
    sv, si = _SORT_CALL(o0, o1)
    idx = si.reshape(_NCAND)
    g = _sc_gather(table, idx)
    g0 = g[:_TOPC, :4].T
    g1 = g[_TOPC:, :4].T
    out5 = _POST_CALL(sv, g0, g1)
    return out5.T[:_POST]


# speedup vs baseline: 1.0488x; 1.0167x over previous
import functools

import jax
import jax.numpy as jnp
from jax import lax
from jax.experimental import pallas as pl
from jax.experimental.pallas import tpu as pltpu
from jax.experimental.pallas import tpu_sc as plsc

_NL0, _NL1 = 16000, 4000
_P0, _P1 = 16384, 4096
_K = 2000
_TOPC = 2048
_OUTP = 1024
_POST = 1000
_IOU_T = 0.7
_MIN_SZ = 0.001
_IMG = 800.0
_BLK = 256
_NCAND = 2 * _TOPC
_TBL = _P0 + _P1
_TBLD = 128
_NW = 32
_BPW = _NCAND // _NW


def _partner(x, d):
    r, c = x.shape
    if d < c:
        lo = jnp.roll(x, -d, axis=1)
        hi = jnp.roll(x, d, axis=1)
        col = jax.lax.broadcasted_iota(jnp.int32, x.shape, 1)
        return jnp.where((col & d) == 0, lo, hi)
    dr = d // c
    lo = jnp.roll(x, -dr, axis=0)
    hi = jnp.roll(x, dr, axis=0)
    row = jax.lax.broadcasted_iota(jnp.int32, x.shape, 0)
    return jnp.where((row & dr) == 0, lo, hi)


def _cmp_swap(kv, ki, extras, d, take_winner):
    pv = _partner(kv, d)
    pi = _partner(ki, d)
    self_first = (kv > pv) | ((kv == pv) & (ki < pi))
    keep_self = self_first == take_winner
    outv = jnp.where(keep_self, kv, pv)
    outi = jnp.where(keep_self, ki, pi)
    outx = [jnp.where(keep_self, e, _partner(e, d)) for e in extras]
    return outv, outi, outx


def _flat_iota(shape):
    r = jax.lax.broadcasted_iota(jnp.int32, shape, 0)
    c = jax.lax.broadcasted_iota(jnp.int32, shape, 1)
    return r * shape[1] + c


def _bitonic_sort_desc(kv, ki, extras=()):
    n = kv.shape[0] * kv.shape[1]
    flat = _flat_iota(kv.shape)
    extras = list(extras)
    k = 2
    while k <= n:
        desc_blk = (flat & k) == 0
        j = k >> 1
        while j:
            lower = (flat & j) == 0
            tw = lower == desc_blk
            kv, ki, extras = _cmp_swap(kv, ki, extras, j, tw)
            j >>= 1
        k <<= 1
    return kv, ki, extras


def _iou_gt(ax1, ay1, ax2, ay2, aarea, bx1, by1, bx2, by2, barea):
    ix1 = jnp.maximum(ax1, bx1)
    iy1 = jnp.maximum(ay1, by1)
    ix2 = jnp.minimum(ax2, bx2)
    iy2 = jnp.minimum(ay2, by2)
    inter = jnp.maximum(ix2 - ix1, 0.0) * jnp.maximum(iy2 - iy1, 0.0)
    iou = inter / (aarea + barea - inter + 1e-9)
    return iou > _IOU_T


def _level_prep(vf, g):
    x1 = jnp.clip(g[0:1, :], 0.0, _IMG)
    y1 = jnp.clip(g[1:2, :], 0.0, _IMG)
    x2 = jnp.clip(g[2:3, :], 0.0, _IMG)
    y2 = jnp.clip(g[3:4, :], 0.0, _IMG)
    score = jax.nn.sigmoid(vf)
    rank = jax.lax.broadcasted_iota(jnp.int32, (1, _TOPC), 1)
    valid = ((x2 - x1 >= _MIN_SZ) & (y2 - y1 >= _MIN_SZ)
             & (score >= 0.0) & (rank < _K))
    ns = jnp.where(valid, score, -jnp.inf)
    area = (x2 - x1) * (y2 - y1)
    return score, x1, y1, x2, y2, rank, ns, area


def _nms_two_levels(lv0, lv1):
    tri = (jax.lax.broadcasted_iota(jnp.int32, (_BLK, _BLK), 0)
           < jax.lax.broadcasted_iota(jnp.int32, (_BLK, _BLK), 1))
    zblk = jnp.zeros((_BLK, _BLK), jnp.float32)
    sups = [jnp.zeros((1, _TOPC), jnp.bool_), jnp.zeros((1, _TOPC), jnp.bool_)]
    kept_blocks = [[], []]
    for b in range(_TOPC // _BLK):
        sl = slice(b * _BLK, (b + 1) * _BLK)
        s_fs, valids, txs = [], [], []
        for li, lv in enumerate((lv0, lv1)):
            _, x1, y1, x2, y2, _, ns, area = lv
            valids.append((ns[:, sl] > -jnp.inf) & ~sups[li][:, sl])
            bx1, by1 = x1[:, sl], y1[:, sl]
            bx2, by2 = x2[:, sl], y2[:, sl]
            ba = area[:, sl]
            tx = (bx1.reshape(_BLK, 1), by1.reshape(_BLK, 1),
                  bx2.reshape(_BLK, 1), by2.reshape(_BLK, 1),
                  ba.reshape(_BLK, 1))
            txs.append(tx)
            s_blk = _iou_gt(*tx, bx1, by1, bx2, by2, ba) & tri
            s_fs.append(s_blk.astype(jnp.float32))
        s_j = jnp.concatenate(
            [jnp.concatenate([s_fs[0], zblk], axis=1),
             jnp.concatenate([zblk, s_fs[1]], axis=1)], axis=0)
        valid_j = jnp.concatenate(valids, axis=1)

        def cond(st):
            return st[1]

        def body(st):
            kp = st[0]
            supv = jnp.dot(kp, s_j, preferred_element_type=jnp.float32) > 0.5
            kn = (valid_j & ~supv).astype(jnp.float32)
            return kn, jnp.any(kn != kp)

        keep_f, _ = jax.lax.while_loop(
            cond, body, (valid_j.astype(jnp.float32), True))
        lo = (b + 1) * _BLK
        for li, lv in enumerate((lv0, lv1)):
            kf = keep_f[:, li * _BLK:(li + 1) * _BLK]
            kept_blocks[li].append(kf > 0.5)
            if lo < _TOPC:
                _, x1, y1, x2, y2, _, _, area = lv
                rest = slice(lo, _TOPC)
                hits = _iou_gt(*txs[li], x1[:, rest], y1[:, rest],
                               x2[:, rest], y2[:, rest], area[:, rest])
                supadd = jnp.dot(kf, hits.astype(jnp.float32),
                                 preferred_element_type=jnp.float32) > 0.5
                pre = jnp.zeros((1, lo), jnp.bool_)
                sups[li] = sups[li] | jnp.concatenate([pre, supadd], axis=1)
    return (jnp.concatenate(kept_blocks[0], axis=1),
            jnp.concatenate(kept_blocks[1], axis=1))


def _sort_body(obj0_ref, obj1_ref, sv_ref, si_ref):
    sv0, si0, _ = _bitonic_sort_desc(obj0_ref[...], _flat_iota(obj0_ref.shape))
    sv1, si1, _ = _bitonic_sort_desc(obj1_ref[...], _flat_iota(obj1_ref.shape))
    sv_ref[...] = jnp.concatenate([sv0[0:1, :], sv1[0:1, :]], axis=0)
    si_ref[...] = jnp.concatenate([si0[0:1, :], si1[0:1, :] + _P0], axis=0)


_SORT_CALL = pl.pallas_call(
    _sort_body,
    out_shape=(jax.ShapeDtypeStruct((2, _TOPC), jnp.float32),
               jax.ShapeDtypeStruct((2, _TOPC), jnp.int32)),
)


@functools.partial(
    pl.kernel,
    mesh=plsc.VectorSubcoreMesh(core_axis_name="c", subcore_axis_name="s"),
    out_type=jax.ShapeDtypeStruct((_NCAND, _TBLD), jnp.float32),
    scratch_types=[
        pltpu.VMEM((_BPW,), jnp.int32),
        pltpu.VMEM((_BPW, _TBLD), jnp.float32),
        pltpu.SemaphoreType.DMA,
    ],
)
def _sc_gather(table_hbm, idx_hbm, out_hbm, idx_v, rows_v, sem):
    wid = lax.axis_index("s") * 2 + lax.axis_index("c")
    base = wid * _BPW
    pltpu.sync_copy(idx_hbm.at[pl.ds(base, _BPW)], idx_v)
    pltpu.async_copy(table_hbm.at[idx_v], rows_v, sem).wait()
    pltpu.sync_copy(rows_v, out_hbm.at[pl.ds(base, _BPW)])


def _post_body(sv_ref, g0_ref, g1_ref, out_ref):
    sv = sv_ref[...]
    lv0 = _level_prep(sv[0:1, :], g0_ref[...])
    lv1 = _level_prep(sv[1:2, :], g1_ref[...])
    k0, k1 = _nms_two_levels(lv0, lv1)
    s0, x10, y10, x20, y20, r0 = lv0[:6]
    s1, x11, y11, x21, y21, r1 = lv1[:6]

    v0 = jnp.where(k0, s0, -jnp.inf)
    v1 = jnp.where(k1, s1, -jnp.inf)

    def cat(a, b):
        return jnp.concatenate([a, b], axis=0)

    mv = cat(v0, v1)
    mv, mi, _ = _bitonic_sort_desc(mv, _flat_iota(mv.shape))

    top_id = mi[0:1, :_OUTP]
    payload = [cat(x10, x11), cat(y10, y11), cat(x20, x21), cat(y20, y21),
               cat(s0, s1)]
    acc = jnp.zeros((5, _OUTP), jnp.float32)
    q = jax.lax.broadcasted_iota(jnp.int32, (_TOPC, _OUTP), 0)
    for r in range(2):
        oh = (q + r * _TOPC == top_id).astype(jnp.float32)
        pr = jnp.concatenate([p[r:r + 1, :] for p in payload], axis=0)
        acc = acc + jnp.dot(pr, oh, preferred_element_type=jnp.float32,
                            precision=jax.lax.Precision.HIGHEST)

    ok = mv[0:1, :_OUTP] > -jnp.inf
    out_ref[...] = jnp.where(ok, acc, 0.0)


_POST_CALL = pl.pallas_call(
    _post_body,
    out_shape=jax.ShapeDtypeStruct((5, _OUTP), jnp.float32),
)


def kernel(proposals, objectness):
    objectness = jax.lax.stop_gradient(objectness)
    o0 = jnp.pad(objectness[:_NL0], (0, _P0 - _NL0),
                 constant_values=-jnp.inf).reshape(8, _P0 // 8)
    o1 = jnp.pad(objectness[_NL0:], (0, _P1 - _NL1),
                 constant_values=-jnp.inf).reshape(2, _P1 // 2)
    table = jnp.concatenate(
        [jnp.pad(proposals[:_NL0], ((0, _P0 - _NL0), (0, 0))),
         jnp.pad(proposals[_NL0:], ((0, _P1 - _NL1), (0, 0)))], axis=0)
    table = jnp.pad(table, ((0, 0), (0, _TBLD - 4)))
    sv, si = _SORT_CALL(o0, o1)
    idx = si.reshape(_NCAND)
    g = _sc_gather(table, idx)
    g0 = g[:_TOPC, :4].T
    g1 = g[_TOPC:, :4].T
    out5 = _POST_CALL(sv, g0, g1)
    return out5.T[:_POST]
